# baseline (speedup 1.0000x reference)
import jax
import jax.numpy as jnp
from jax.experimental import pallas as pl
from jax.experimental.pallas import tpu as pltpu

NEG = -100000000.0
EPS = 1.1920929e-07
BIGI = 2**30

NCHUNK = 2


def _chunk_compute(vml, vmm_i, pp, un):
    x = jnp.where(vmm_i != 0, NEG, vml)
    m = jnp.max(x, axis=1, keepdims=True)
    e = jnp.exp(x - m)
    s = jnp.sum(e, axis=1, keepdims=True)
    sxe = jnp.sum(e * x, axis=1, keepdims=True)
    logs = jnp.log(s)
    lse = m + logs
    vm_lp = -logs[:, 0]
    vm_ent = jnp.where(m[:, 0] == NEG, 0.0, lse[:, 0] - sxe[:, 0] / s[:, 0])
    ii = jax.lax.broadcasted_iota(jnp.int32, x.shape, 1)
    sel_vm = jnp.min(jnp.where(x == m, ii, BIGI), axis=1)

    p = pp * un
    S = jnp.sum(p, axis=1, keepdims=True)
    cnt = jnp.sum(un, axis=1, keepdims=True)
    small = S < 0.0001
    p2 = jnp.where(small, un, p)
    S2 = jnp.where(small, cnt, S)
    q = p2 / S2
    lq = jnp.log(jnp.clip(q, EPS, 1.0 - EPS))
    pm_ent = -jnp.sum(lq * q, axis=1)
    mq = jnp.max(q, axis=1, keepdims=True)
    jj = jax.lax.broadcasted_iota(jnp.int32, q.shape, 1)
    sel_pm = jnp.min(jnp.where(q == mq, jj, BIGI), axis=1)
    pm_lp = jnp.log(jnp.clip(mq[:, 0], EPS, 1.0 - EPS))
    return sel_vm, sel_pm, vm_lp + pm_lp, vm_ent + pm_ent


def _heads_kernel(vml_hbm, vmm_hbm, pmp_hbm, pmm_hbm,
                  selvm_ref, selpm_ref, lp_ref, ent_ref,
                  xb0, xb1, mb0, mb1, pb0, pb1, qb0, qb1, sems):
    xb = (xb0, xb1)
    mb = (mb0, mb1)
    pb = (pb0, pb1)
    qb = (qb0, qb1)
    cr = xb0.shape[0]

    def start(c):
        buf = c % 2
        sl = pl.ds(c * cr, cr)
        cps = (
            pltpu.make_async_copy(vml_hbm.at[sl, :], xb[buf], sems.at[buf, 0]),
            pltpu.make_async_copy(vmm_hbm.at[sl, :], mb[buf], sems.at[buf, 1]),
            pltpu.make_async_copy(pmp_hbm.at[sl, :], pb[buf], sems.at[buf, 2]),
            pltpu.make_async_copy(pmm_hbm.at[sl, :], qb[buf], sems.at[buf, 3]),
        )
        for cp in cps:
            cp.start()
        return cps

    pending = start(0)
    for c in range(NCHUNK):
        for cp in pending:
            cp.wait()
        if c + 1 < NCHUNK:
            pending = start(c + 1)
        buf = c % 2
        sel_vm, sel_pm, lp, ent = _chunk_compute(
            xb[buf][...],
            mb[buf][...].astype(jnp.int32),
            pb[buf][...],
            1.0 - qb[buf][...].astype(jnp.float32),
        )
        sl = pl.ds(c * cr, cr)
        selvm_ref[sl] = sel_vm
        selpm_ref[sl] = sel_pm
        lp_ref[sl] = lp
        ent_ref[sl] = ent


def kernel(vm_logits, vm_mask, pm_probs, pm_mask):
    B = vm_logits.shape[0]
    NV = vm_logits.shape[1]
    NP = pm_probs.shape[1]
    CR = B // NCHUNK
    out = pl.pallas_call(
        _heads_kernel,
        in_specs=[pl.BlockSpec(memory_space=pl.ANY)] * 4,
        out_shape=(
            jax.ShapeDtypeStruct((B,), jnp.int32),
            jax.ShapeDtypeStruct((B,), jnp.int32),
            jax.ShapeDtypeStruct((B,), jnp.float32),
            jax.ShapeDtypeStruct((B,), jnp.float32),
        ),
        scratch_shapes=[
            pltpu.VMEM((CR, NV), jnp.float32),
            pltpu.VMEM((CR, NV), jnp.float32),
            pltpu.VMEM((CR, NV), jnp.int8),
            pltpu.VMEM((CR, NV), jnp.int8),
            pltpu.VMEM((CR, NP), jnp.float32),
            pltpu.VMEM((CR, NP), jnp.float32),
            pltpu.VMEM((CR, NP), jnp.int8),
            pltpu.VMEM((CR, NP), jnp.int8),
            pltpu.SemaphoreType.DMA((2, 4)),
        ],
    )(vm_logits, vm_mask.view(jnp.int8), pm_probs, pm_mask.view(jnp.int8))
    return out
